# Initial kernel scaffold; baseline (speedup 1.0000x reference)
import jax
import jax.numpy as jnp
from jax.experimental import pallas as pl


def kernel(input, adj, assignments, weight):
    raise NotImplementedError("write your pallas kernel here")


# speedup vs baseline: 1.0463x; 1.0463x over previous
import jax
import jax.numpy as jnp
from jax.experimental import pallas as pl
from jax.experimental.pallas import tpu as pltpu

BM = 512


def _body(x_ref, adj_ref, asg_ref, w_ref, out1_ref, out2_ref,
          sup_n_ref, sup_c_ref):
    i = pl.program_id(0)
    n = sup_n_ref.shape[0]

    @pl.when(i == 0)
    def _prologue():
        w = w_ref[...]
        sup_n = jnp.dot(x_ref[:n, :], w, preferred_element_type=jnp.float32)
        sup_n_ref[...] = sup_n
        sup_c_ref[...] = jnp.dot(x_ref[n:, :], w,
                                 preferred_element_type=jnp.float32)
        asg = asg_ref[...]
        colsum = jnp.sum(asg, axis=0)
        out2 = jax.lax.dot_general(
            asg, sup_n, (((0,), (0,)), ((), ())),
            preferred_element_type=jnp.float32)
        out2_ref[...] = out2 / colsum[:, None]

    a_blk = asg_ref[pl.ds(i * BM, BM), :]
    a_norm = a_blk / jnp.sum(a_blk, axis=1, keepdims=True)
    out1_ref[...] = (
        jnp.dot(adj_ref[...], sup_n_ref[...],
                preferred_element_type=jnp.float32)
        + jnp.dot(a_norm, sup_c_ref[...], preferred_element_type=jnp.float32))


def kernel(input, adj, assignments, weight):
    n, c = assignments.shape
    in_f = input.shape[1]
    out_f = weight.shape[1]
    grid = (n // BM,)

    out1, out2 = pl.pallas_call(
        _body,
        grid=grid,
        in_specs=[
            pl.BlockSpec((n + c, in_f), lambda i: (0, 0)),
            pl.BlockSpec((BM, n), lambda i: (i, 0)),
            pl.BlockSpec((n, c), lambda i: (0, 0)),
            pl.BlockSpec((in_f, out_f), lambda i: (0, 0)),
        ],
        out_specs=[
            pl.BlockSpec((BM, out_f), lambda i: (i, 0)),
            pl.BlockSpec((c, out_f), lambda i: (0, 0)),
        ],
        out_shape=[
            jax.ShapeDtypeStruct((n, out_f), jnp.float32),
            jax.ShapeDtypeStruct((c, out_f), jnp.float32),
        ],
        scratch_shapes=[
            pltpu.VMEM((n, out_f), jnp.float32),
            pltpu.VMEM((c, out_f), jnp.float32),
        ],
    )(input, adj, assignments, weight)

    return jnp.concatenate((out1, out2), axis=0)


# speedup vs baseline: 1.0530x; 1.0064x over previous
import jax
import jax.numpy as jnp
from jax.experimental import pallas as pl
from jax.experimental.pallas import tpu as pltpu

BM = 512


def _body(x_ref, adj_ref, asg_ref, w_ref, out1_ref, out2_ref,
          sup_n_ref, sup_c_ref):
    i = pl.program_id(0)
    n = sup_n_ref.shape[0]

    @pl.when(i == 0)
    def _prologue():
        w = w_ref[...]
        sup_n = jnp.dot(x_ref[:n, :], w, preferred_element_type=jnp.float32)
        sup_n_ref[...] = sup_n.astype(jnp.bfloat16)
        sup_c_ref[...] = jnp.dot(x_ref[n:, :], w,
                                 preferred_element_type=jnp.float32)
        asg = asg_ref[...]
        colsum = jnp.sum(asg, axis=0)
        out2 = jax.lax.dot_general(
            asg, sup_n, (((0,), (0,)), ((), ())),
            preferred_element_type=jnp.float32)
        out2_ref[...] = out2 / colsum[:, None]

    a_blk = asg_ref[pl.ds(i * BM, BM), :]
    a_norm = a_blk / jnp.sum(a_blk, axis=1, keepdims=True)
    out1_ref[...] = (
        jnp.dot(adj_ref[...].astype(jnp.bfloat16), sup_n_ref[...],
                preferred_element_type=jnp.float32)
        + jnp.dot(a_norm, sup_c_ref[...], preferred_element_type=jnp.float32))


def kernel(input, adj, assignments, weight):
    n, c = assignments.shape
    in_f = input.shape[1]
    out_f = weight.shape[1]
    grid = (n // BM,)

    out1, out2 = pl.pallas_call(
        _body,
        grid=grid,
        in_specs=[
            pl.BlockSpec((n + c, in_f), lambda i: (0, 0)),
            pl.BlockSpec((BM, n), lambda i: (i, 0)),
            pl.BlockSpec((n, c), lambda i: (0, 0)),
            pl.BlockSpec((in_f, out_f), lambda i: (0, 0)),
        ],
        out_specs=[
            pl.BlockSpec((BM, out_f), lambda i: (i, 0)),
            pl.BlockSpec((c, out_f), lambda i: (0, 0)),
        ],
        out_shape=[
            jax.ShapeDtypeStruct((n, out_f), jnp.float32),
            jax.ShapeDtypeStruct((c, out_f), jnp.float32),
        ],
        scratch_shapes=[
            pltpu.VMEM((n, out_f), jnp.bfloat16),
            pltpu.VMEM((c, out_f), jnp.float32),
        ],
    )(input, adj, assignments, weight)

    return jnp.concatenate((out1, out2), axis=0)


# speedup vs baseline: 1.1983x; 1.1380x over previous
import jax
import jax.numpy as jnp
from jax.experimental import pallas as pl
from jax.experimental.pallas import tpu as pltpu

BM = 512


def _body(x_ref, adj_ref, asg_ref, w_ref, out_ref, sup_n_ref, sup_c_ref):
    i = pl.program_id(0)
    nblk = pl.num_programs(0) - 1
    n = sup_n_ref.shape[0]
    c = sup_c_ref.shape[0]

    @pl.when(i == 0)
    def _prologue():
        w = w_ref[...]
        sup_n_ref[...] = jnp.dot(x_ref[:n, :], w,
                                 preferred_element_type=jnp.float32)
        sup_c_ref[...] = jnp.dot(x_ref[n:, :], w,
                                 preferred_element_type=jnp.float32)

    @pl.when(i < nblk)
    def _out1_block():
        a_blk = asg_ref[pl.ds(i * BM, BM), :]
        a_norm = a_blk / jnp.sum(a_blk, axis=1, keepdims=True)
        out_ref[...] = (
            jnp.dot(adj_ref[...], sup_n_ref[...],
                    preferred_element_type=jnp.float32)
            + jnp.dot(a_norm, sup_c_ref[...],
                      preferred_element_type=jnp.float32))

    @pl.when(i == nblk)
    def _out2_tail():
        asg = asg_ref[...]
        colsum = jnp.sum(asg, axis=0)
        out2 = jax.lax.dot_general(
            asg, sup_n_ref[...], (((0,), (0,)), ((), ())),
            preferred_element_type=jnp.float32)
        out_ref[pl.ds(0, c), :] = out2 / colsum[:, None]


def kernel(input, adj, assignments, weight):
    n, c = assignments.shape
    in_f = input.shape[1]
    out_f = weight.shape[1]
    nblk = n // BM
    grid = (nblk + 1,)

    return pl.pallas_call(
        _body,
        grid=grid,
        in_specs=[
            pl.BlockSpec((n + c, in_f), lambda i: (0, 0)),
            pl.BlockSpec((BM, n), lambda i: (jnp.minimum(i, nblk - 1), 0)),
            pl.BlockSpec((n, c), lambda i: (0, 0)),
            pl.BlockSpec((in_f, out_f), lambda i: (0, 0)),
        ],
        out_specs=pl.BlockSpec((BM, out_f), lambda i: (i, 0)),
        out_shape=jax.ShapeDtypeStruct((n + c, out_f), jnp.float32),
        scratch_shapes=[
            pltpu.VMEM((n, out_f), jnp.float32),
            pltpu.VMEM((c, out_f), jnp.float32),
        ],
    )(input, adj, assignments, weight)
